# baseline (speedup 1.0000x reference)
import jax
import jax.numpy as jnp
from jax import lax
from jax.experimental import pallas as pl
from jax.experimental.pallas import tpu as pltpu

F32 = jnp.float32
F8 = jnp.float8_e4m3fn
LANE = 128
VMEM_LIMIT = 60 * 1024 * 1024

_ONE_F8_BITS = 0x38


def _prelu(x, a):
    return jnp.where(x > 0, x, a * x)


def _hi_lo(x):
    hi = x.astype(F8)
    lo = (x - hi.astype(F32)).astype(F8)
    return hi, lo


def _proj_kernel(h_ref, w1_ref, w2_ref, a_ref, xwa_ref, selfw_ref):
    a = a_ref[0]
    xw = jnp.dot(h_ref[...], w1_ref[...], preferred_element_type=F32)
    hi, lo = _hi_lo(xw)
    xwa_ref[:, 0:LANE] = hi
    xwa_ref[:, LANE:2 * LANE] = lo
    xwa_ref[:, 2 * LANE:2 * LANE + 8] = jnp.ones(
        (xw.shape[0], 8), F8)
    selfw_ref[...] = jnp.dot(_prelu(xw, a), w2_ref[...],
                             preferred_element_type=F32)


def _proj(h, w1, w2, a_arr, tn):
    n = h.shape[0]
    return pl.pallas_call(
        _proj_kernel,
        out_shape=(jax.ShapeDtypeStruct((n, 2 * LANE + 8), F8),
                   jax.ShapeDtypeStruct((n, LANE), F32)),
        grid=(n // tn,),
        in_specs=[
            pl.BlockSpec((tn, LANE), lambda i: (i, 0)),
            pl.BlockSpec((LANE, LANE), lambda i: (0, 0)),
            pl.BlockSpec((LANE, LANE), lambda i: (0, 0)),
            pl.BlockSpec(memory_space=pltpu.MemorySpace.SMEM),
        ],
        out_specs=(pl.BlockSpec((tn, 2 * LANE + 8), lambda i: (i, 0)),
                   pl.BlockSpec((tn, LANE), lambda i: (i, 0))),
        compiler_params=pltpu.CompilerParams(
            dimension_semantics=("parallel",),
            vmem_limit_bytes=VMEM_LIMIT),
    )(h, w1, w2, a_arr)


def _n2e_kernel(xwa_ref, h8_ref, w2_ref, a_ref, ewa_ref, acc_ref):
    k = pl.program_id(1)

    @pl.when(k == 0)
    def _():
        acc_ref[...] = jnp.zeros_like(acc_ref)

    acc_ref[...] += lax.dot_general(
        xwa_ref[...], h8_ref[...],
        dimension_numbers=(((0,), (0,)), ((), ())),
        preferred_element_type=F32)

    @pl.when(k == pl.num_programs(1) - 1)
    def _():
        a = a_ref[0]
        de = acc_ref[2 * LANE:2 * LANE + 1, :]
        de_inv = jnp.where(de > 0, 1.0 / de, 0.0)
        e_t = _prelu((acc_ref[0:LANE, :] + acc_ref[LANE:2 * LANE, :]) * de_inv,
                     a)
        ew = lax.dot_general(e_t, w2_ref[...],
                             dimension_numbers=(((0,), (0,)), ((), ())),
                             preferred_element_type=F32)
        hi, lo = _hi_lo(ew)
        ewa_ref[:, 0:LANE] = hi
        ewa_ref[:, LANE:2 * LANE] = lo
        ewa_ref[:, 2 * LANE:2 * LANE + 8] = jnp.ones(
            (ew.shape[0], 8), F8)


def _n2e(xwa, h8, w2, a_arr, tn, te):
    n, m = h8.shape
    return pl.pallas_call(
        _n2e_kernel,
        out_shape=jax.ShapeDtypeStruct((m, 2 * LANE + 8), F8),
        grid=(m // te, n // tn),
        in_specs=[
            pl.BlockSpec((tn, 2 * LANE + 8), lambda j, k: (k, 0)),
            pl.BlockSpec((tn, te), lambda j, k: (k, j)),
            pl.BlockSpec((LANE, LANE), lambda j, k: (0, 0)),
            pl.BlockSpec(memory_space=pltpu.MemorySpace.SMEM),
        ],
        out_specs=pl.BlockSpec((te, 2 * LANE + 8), lambda j, k: (j, 0)),
        scratch_shapes=[pltpu.VMEM((2 * LANE + 8, te), F32)],
        compiler_params=pltpu.CompilerParams(
            dimension_semantics=("parallel", "arbitrary"),
            vmem_limit_bytes=VMEM_LIMIT),
    )(xwa, h8, w2, a_arr)


def _e2n_kernel(h8_ref, ewa_ref, selfw_ref, a_ref, y_ref, acc_ref):
    k = pl.program_id(1)

    @pl.when(k == 0)
    def _():
        acc_ref[...] = jnp.zeros_like(acc_ref)

    acc_ref[...] += jnp.dot(h8_ref[...], ewa_ref[...],
                            preferred_element_type=F32)

    @pl.when(k == pl.num_programs(1) - 1)
    def _():
        a = a_ref[0]
        dn_inv = 1.0 / (acc_ref[:, 2 * LANE:2 * LANE + 1] + 1.0)
        s = (acc_ref[:, 0:LANE] + acc_ref[:, LANE:2 * LANE] + selfw_ref[...])
        y_ref[...] = _prelu(s * dn_inv, a)


def _e2n(h8, ewa, selfw, a_arr, tn, te):
    n, m = h8.shape
    return pl.pallas_call(
        _e2n_kernel,
        out_shape=jax.ShapeDtypeStruct((n, LANE), F32),
        grid=(n // tn, m // te),
        in_specs=[
            pl.BlockSpec((tn, te), lambda i, k: (i, k)),
            pl.BlockSpec((te, 2 * LANE + 8), lambda i, k: (k, 0)),
            pl.BlockSpec((tn, LANE), lambda i, k: (i, 0)),
            pl.BlockSpec(memory_space=pltpu.MemorySpace.SMEM),
        ],
        out_specs=pl.BlockSpec((tn, LANE), lambda i, k: (i, 0)),
        scratch_shapes=[pltpu.VMEM((tn, 2 * LANE + 8), F32)],
        compiler_params=pltpu.CompilerParams(
            dimension_semantics=("parallel", "arbitrary"),
            vmem_limit_bytes=VMEM_LIMIT),
    )(h8, ewa, selfw, a_arr)


def _forward(x, hyperedge_index, convs, prelu, num_nodes, num_edges,
             tn_a=4096, tn_b=2048, te_b=4096, tn_c=4096, te_c=2048):
    N, F = x.shape
    M = num_edges
    assert N == num_nodes and F == LANE

    keys = jnp.sort(hyperedge_index[0] * M + hyperedge_index[1])
    h8 = jnp.zeros((N * M,), F8).at[keys].set(
        jnp.asarray(1.0, F8), indices_are_sorted=True).reshape(N, M)

    a_arr = jnp.full((1,), prelu, F32)

    h = x
    for (w1, w2) in convs:
        xwa, selfw = _proj(h, w1, w2, a_arr, tn_a)
        ewa = _n2e(xwa, h8, w2, a_arr, tn_b, te_b)
        h = _e2n(h8, ewa, selfw, a_arr, tn_c, te_c)
    return h


def kernel(x, hyperedge_index, w1_0, w2_0, w1_1, w2_1, prelu):
    return _forward(x, hyperedge_index, ((w1_0, w2_0), (w1_1, w2_1)), prelu,
                    num_nodes=32768, num_edges=8192)


# speedup vs baseline: 1.2056x; 1.2056x over previous
import jax
import jax.numpy as jnp
from jax import lax
from jax.experimental import pallas as pl
from jax.experimental.pallas import tpu as pltpu

F32 = jnp.float32
F8 = jnp.float8_e4m3fn
LANE = 128
AUG = 2 * LANE + 8
VMEM_LIMIT = 60 * 1024 * 1024

_ONE_F8_BITS = 0x38


def _prelu(x, a):
    return jnp.where(x > 0, x, a * x)


def _hi_lo(x):
    hi = x.astype(F8)
    lo = (x - hi.astype(F32)).astype(F8)
    return hi, lo


def _proj_kernel(h_ref, w1_ref, w2_ref, a_ref, xwa_ref, selfw_ref):
    a = a_ref[0]
    xw_t = lax.dot_general(w1_ref[...], h_ref[...],
                           dimension_numbers=(((0,), (1,)), ((), ())),
                           preferred_element_type=F32)
    hi, lo = _hi_lo(xw_t)
    xwa_ref[0:LANE, :] = hi
    xwa_ref[LANE:2 * LANE, :] = lo
    xwa_ref[2 * LANE:AUG, :] = jnp.ones((8, xw_t.shape[1]), F8)
    selfw_ref[...] = lax.dot_general(
        _prelu(xw_t, a), w2_ref[...],
        dimension_numbers=(((0,), (0,)), ((), ())),
        preferred_element_type=F32)


def _proj(h, w1, w2, a_arr, tn):
    n = h.shape[0]
    return pl.pallas_call(
        _proj_kernel,
        out_shape=(jax.ShapeDtypeStruct((AUG, n), F8),
                   jax.ShapeDtypeStruct((n, LANE), F32)),
        grid=(n // tn,),
        in_specs=[
            pl.BlockSpec((tn, LANE), lambda i: (i, 0)),
            pl.BlockSpec((LANE, LANE), lambda i: (0, 0)),
            pl.BlockSpec((LANE, LANE), lambda i: (0, 0)),
            pl.BlockSpec(memory_space=pltpu.MemorySpace.SMEM),
        ],
        out_specs=(pl.BlockSpec((AUG, tn), lambda i: (0, i)),
                   pl.BlockSpec((tn, LANE), lambda i: (i, 0))),
        compiler_params=pltpu.CompilerParams(
            dimension_semantics=("parallel",),
            vmem_limit_bytes=VMEM_LIMIT),
    )(h, w1, w2, a_arr)


def _n2e_kernel(xwa_ref, h8_ref, w2_ref, a_ref, ewa_ref):
    a = a_ref[0]
    acc = lax.dot_general(xwa_ref[...], h8_ref[...],
                          dimension_numbers=(((1,), (0,)), ((), ())),
                          preferred_element_type=F32)
    de = acc[2 * LANE:2 * LANE + 1, :]
    de_inv = jnp.where(de > 0, 1.0 / de, 0.0)
    e_t = _prelu((acc[0:LANE, :] + acc[LANE:2 * LANE, :]) * de_inv, a)
    ew = lax.dot_general(e_t, w2_ref[...],
                         dimension_numbers=(((0,), (0,)), ((), ())),
                         preferred_element_type=F32)
    hi, lo = _hi_lo(ew)
    ewa_ref[:, 0:LANE] = hi
    ewa_ref[:, LANE:2 * LANE] = lo
    ewa_ref[:, 2 * LANE:AUG] = jnp.ones((ew.shape[0], 8), F8)


def _n2e(xwa, h8, w2, a_arr, te):
    n, m = h8.shape
    return pl.pallas_call(
        _n2e_kernel,
        out_shape=jax.ShapeDtypeStruct((m, AUG), F8),
        grid=(m // te,),
        in_specs=[
            pl.BlockSpec((AUG, n), lambda j: (0, 0)),
            pl.BlockSpec((n, te), lambda j: (0, j)),
            pl.BlockSpec((LANE, LANE), lambda j: (0, 0)),
            pl.BlockSpec(memory_space=pltpu.MemorySpace.SMEM),
        ],
        out_specs=pl.BlockSpec((te, AUG), lambda j: (j, 0)),
        compiler_params=pltpu.CompilerParams(
            dimension_semantics=("parallel",),
            vmem_limit_bytes=VMEM_LIMIT),
    )(xwa, h8, w2, a_arr)


def _e2n_kernel(h8_ref, ewa_ref, selfw_ref, a_ref, y_ref):
    a = a_ref[0]
    acc = jnp.dot(h8_ref[...], ewa_ref[...],
                  preferred_element_type=F32)
    dn_inv = 1.0 / (acc[:, 2 * LANE:2 * LANE + 1] + 1.0)
    s = acc[:, 0:LANE] + acc[:, LANE:2 * LANE] + selfw_ref[...]
    y_ref[...] = _prelu(s * dn_inv, a)


def _e2n(h8, ewa, selfw, a_arr, tn):
    n, m = h8.shape
    return pl.pallas_call(
        _e2n_kernel,
        out_shape=jax.ShapeDtypeStruct((n, LANE), F32),
        grid=(n // tn,),
        in_specs=[
            pl.BlockSpec((tn, m), lambda i: (i, 0)),
            pl.BlockSpec((m, AUG), lambda i: (0, 0)),
            pl.BlockSpec((tn, LANE), lambda i: (i, 0)),
            pl.BlockSpec(memory_space=pltpu.MemorySpace.SMEM),
        ],
        out_specs=pl.BlockSpec((tn, LANE), lambda i: (i, 0)),
        compiler_params=pltpu.CompilerParams(
            dimension_semantics=("parallel",),
            vmem_limit_bytes=VMEM_LIMIT),
    )(h8, ewa, selfw, a_arr)


def _forward(x, hyperedge_index, convs, prelu, num_nodes, num_edges,
             tn_a=4096, te_b=512, tn_c=1024):
    N, F = x.shape
    M = num_edges
    assert N == num_nodes and F == LANE

    keys = jnp.sort(hyperedge_index[0] * M + hyperedge_index[1])
    h8 = jnp.zeros((N * M,), jnp.uint8).at[keys].set(
        jnp.uint8(_ONE_F8_BITS), indices_are_sorted=True)
    h8 = lax.bitcast_convert_type(h8, F8).reshape(N, M)

    a_arr = jnp.full((1,), prelu, F32)

    h = x
    for (w1, w2) in convs:
        xwa, selfw = _proj(h, w1, w2, a_arr, tn_a)
        ewa = _n2e(xwa, h8, w2, a_arr, te_b)
        h = _e2n(h8, ewa, selfw, a_arr, tn_c)
    return h


def kernel(x, hyperedge_index, w1_0, w2_0, w1_1, w2_1, prelu):
    return _forward(x, hyperedge_index, ((w1_0, w2_0), (w1_1, w2_1)), prelu,
                    num_nodes=32768, num_edges=8192)
